# Initial kernel scaffold; baseline (speedup 1.0000x reference)
import functools

import jax
import jax.numpy as jnp
from jax import lax
from jax.experimental import pallas as pl
from jax.experimental.pallas import tpu as pltpu
from jax.experimental.pallas import tpu_sc as plsc

_N, _E, _D, _G = 10000, 320000, 128, 64
_NC, _NS = 2, 16
_NW = _NC * _NS
_EPW = _E // _NW
_ECH = 80
_NCHUNK = _EPW // _ECH
_RPS = _N // _NS
_RCH = 125
_RNCH = _RPS // _RCH


def _sc_body(feat_h, src_h, dst_h, zero_h, out_h,
             idx_s, idx_d, rows, vbuf, acc, sem):
  c = lax.axis_index("c")
  s = lax.axis_index("s")
  wid = s * _NC + c
  row0 = s * _RPS

  pltpu.sync_copy(zero_h, vbuf)
  for k in range(_RNCH):
    pltpu.sync_copy(vbuf, acc.at[pl.ds(row0 + k * _RCH, _RCH)])
  plsc.subcore_barrier()

  def step(i, carry):
    e0 = wid * _EPW + i * _ECH
    pltpu.sync_copy(src_h.at[pl.ds(e0, _ECH)], idx_s)
    pltpu.sync_copy(dst_h.at[pl.ds(e0, _ECH)], idx_d)
    pltpu.async_copy(feat_h.at[idx_s], rows, sem).wait()
    pltpu.sync_copy(rows, acc.at[idx_d], add=True)
    return carry

  lax.fori_loop(0, _NCHUNK, step, 0)
  plsc.subcore_barrier()

  for k in range(_RNCH):
    sl = pl.ds(row0 + k * _RCH, _RCH)
    pltpu.sync_copy(acc.at[sl], vbuf)
    pltpu.sync_copy(vbuf, out_h.at[c, sl])


_sc_seg_sum = functools.partial(
    pl.kernel,
    out_type=jax.ShapeDtypeStruct((_NC, _N, _D), jnp.float32),
    mesh=plsc.VectorSubcoreMesh(
        core_axis_name="c", subcore_axis_name="s",
        num_cores=_NC, num_subcores=_NS),
    scratch_types=[
        pltpu.VMEM((_ECH,), jnp.int32),
        pltpu.VMEM((_ECH,), jnp.int32),
        pltpu.VMEM((_ECH, _D), jnp.float32),
        pltpu.VMEM((_RCH, _D), jnp.float32),
        pltpu.VMEM_SHARED((_N, _D), jnp.float32),
        pltpu.SemaphoreType.DMA,
    ],
)(_sc_body)


def _gin_mlp(z, W1, b1, g, be, W2, b2):
  h = jnp.dot(z, W1, preferred_element_type=jnp.float32) + b1
  mean = jnp.mean(h, axis=0, keepdims=True)
  var = jnp.mean((h - mean) ** 2, axis=0, keepdims=True)
  h = (h - mean) / jnp.sqrt(var + 1e-5) * g + be
  h = jnp.maximum(h, 0.0)
  h = jnp.dot(h, W2, preferred_element_type=jnp.float32) + b2
  return jnp.maximum(h, 0.0)


def _tc1_body(x, p, W1, b1, g1, be1, W2, b2, h1_out):
  z = x[...] + p[0] + p[1]
  h1_out[...] = _gin_mlp(z, W1[...], b1[...], g1[...], be1[...],
                         W2[...], b2[...])


def _tc2_body(h1, p, W3, b3, g2, be2, W4, b4, Wr1, br1, Wr2, br2, Wr3, br3,
              Wm1, bm1, Wm2, bm2, batch2d, out_o, xrec_o):
  z = h1[...] + p[0] + p[1]
  h2 = _gin_mlp(z, W3[...], b3[...], g2[...], be2[...], W4[...], b4[...])

  r = jnp.maximum(jnp.dot(h2, Wr1[...],
                          preferred_element_type=jnp.float32) + br1[...], 0.0)
  r = jnp.maximum(jnp.dot(r, Wr2[...],
                          preferred_element_type=jnp.float32) + br2[...], 0.0)
  xrec_o[...] = jnp.maximum(
      jnp.dot(r, Wr3[...], preferred_element_type=jnp.float32) + br3[...], 0.0)

  gids = lax.broadcasted_iota(jnp.int32, (_N, _G), 1)
  onehot = (batch2d[...] == gids).astype(jnp.float32)
  pooled = lax.dot_general(onehot, h2, (((0,), (0,)), ((), ())),
                           preferred_element_type=jnp.float32)
  m = jnp.maximum(jnp.dot(pooled, Wm1[...],
                          preferred_element_type=jnp.float32) + bm1[...], 0.0)
  out_o[...] = jnp.dot(m, Wm2[...],
                       preferred_element_type=jnp.float32) + bm2[...]


_tc1 = pl.pallas_call(
    _tc1_body,
    out_shape=jax.ShapeDtypeStruct((_N, _D), jnp.float32),
)

_tc2 = pl.pallas_call(
    _tc2_body,
    out_shape=(
        jax.ShapeDtypeStruct((_G, 64), jnp.float32),
        jax.ShapeDtypeStruct((_N, 4), jnp.float32),
    ),
)


def kernel(x, W1, b1, g1, be1, W2, b2, W3, b3, g2, be2, W4, b4,
           Wr1, br1, Wr2, br2, Wr3, br3, Wm1, bm1, Wm2, bm2,
           edge_index, batch):
  src = edge_index[0]
  dst = edge_index[1]
  zeros_chunk = jnp.zeros((_RCH, _D), jnp.float32)
  row = lambda v: v.reshape(1, -1)

  p1 = _sc_seg_sum(x, src, dst, zeros_chunk)
  h1 = _tc1(x, p1, W1, row(b1), row(g1), row(be1), W2, row(b2))
  p2 = _sc_seg_sum(h1, src, dst, zeros_chunk)
  out, x_rec = _tc2(h1, p2, W3, row(b3), row(g2), row(be2), W4, row(b4),
                    Wr1, row(br1), Wr2, row(br2), Wr3, row(br3),
                    Wm1, row(bm1), Wm2, row(bm2),
                    batch.reshape(_N, 1))
  return (out, x_rec)


# speedup vs baseline: 4.7131x; 4.7131x over previous
import functools

import jax
import jax.numpy as jnp
from jax import lax
from jax.experimental import pallas as pl
from jax.experimental.pallas import tpu as pltpu
from jax.experimental.pallas import tpu_sc as plsc

_N, _E, _D, _G = 10000, 320000, 128, 64
_NC, _NS = 2, 16
_NW = _NC * _NS
_EPW = _E // _NW
_ECH = 80
_NCHUNK = _EPW // _ECH
_NP = 10240
_RPS = _NP // _NS
_RCH = 128
_RNCH = _RPS // _RCH


def _sc_body(feat_h, src_h, dst_h, zero_h, out_h,
             idx_s, idx_d, rows, vbuf, acc, sem):
  c = lax.axis_index("c")
  s = lax.axis_index("s")
  wid = s * _NC + c
  row0 = s * _RPS

  pltpu.sync_copy(zero_h, vbuf)
  for k in range(_RNCH):
    pltpu.sync_copy(vbuf, acc.at[pl.ds(row0 + k * _RCH, _RCH)])
  plsc.subcore_barrier()

  def step(i, carry):
    e0 = wid * _EPW + i * _ECH
    pltpu.sync_copy(src_h.at[pl.ds(e0, _ECH)], idx_s)
    pltpu.sync_copy(dst_h.at[pl.ds(e0, _ECH)], idx_d)
    pltpu.async_copy(feat_h.at[idx_s], rows, sem).wait()
    pltpu.sync_copy(rows, acc.at[idx_d], add=True)
    return carry

  lax.fori_loop(0, _NCHUNK, step, 0)
  plsc.subcore_barrier()

  for k in range(_RNCH):
    sl = pl.ds(row0 + k * _RCH, _RCH)
    pltpu.sync_copy(acc.at[sl], vbuf)
    pltpu.sync_copy(vbuf, out_h.at[c, sl])


@functools.cache
def _sc_seg_sum_fn():
  return pl.kernel(
      _sc_body,
      out_type=jax.ShapeDtypeStruct((_NC, _NP, _D), jnp.float32),
      mesh=plsc.VectorSubcoreMesh(
          core_axis_name="c", subcore_axis_name="s",
          num_cores=_NC, num_subcores=_NS),
      scratch_types=[
          pltpu.VMEM((_ECH,), jnp.int32),
          pltpu.VMEM((_ECH,), jnp.int32),
          pltpu.VMEM((_ECH, _D), jnp.float32),
          pltpu.VMEM((_RCH, _D), jnp.float32),
          pltpu.VMEM_SHARED((_NP, _D), jnp.float32),
          pltpu.SemaphoreType.DMA,
      ],
  )


def _sc_seg_sum(feat, src, dst, zeros_chunk):
  return _sc_seg_sum_fn()(feat, src, dst, zeros_chunk)


def _gin_mlp(z, W1, b1, g, be, W2, b2):
  h = jnp.dot(z, W1, preferred_element_type=jnp.float32) + b1
  mean = jnp.mean(h, axis=0, keepdims=True)
  var = jnp.mean((h - mean) ** 2, axis=0, keepdims=True)
  h = (h - mean) / jnp.sqrt(var + 1e-5) * g + be
  h = jnp.maximum(h, 0.0)
  h = jnp.dot(h, W2, preferred_element_type=jnp.float32) + b2
  return jnp.maximum(h, 0.0)


def _tc1_body(x, p, W1, b1, g1, be1, W2, b2, h1_out):
  z = x[...] + p[0, :_N] + p[1, :_N]
  h1_out[...] = _gin_mlp(z, W1[...], b1[...], g1[...], be1[...],
                         W2[...], b2[...])


def _tc2_body(h1, p, W3, b3, g2, be2, W4, b4, Wr1, br1, Wr2, br2, Wr3, br3,
              Wm1, bm1, Wm2, bm2, batch2d, out_o, xrec_o):
  z = h1[...] + p[0, :_N] + p[1, :_N]
  h2 = _gin_mlp(z, W3[...], b3[...], g2[...], be2[...], W4[...], b4[...])

  r = jnp.maximum(jnp.dot(h2, Wr1[...],
                          preferred_element_type=jnp.float32) + br1[...], 0.0)
  r = jnp.maximum(jnp.dot(r, Wr2[...],
                          preferred_element_type=jnp.float32) + br2[...], 0.0)
  xrec_o[...] = jnp.maximum(
      jnp.dot(r, Wr3[...], preferred_element_type=jnp.float32) + br3[...], 0.0)

  gids = lax.broadcasted_iota(jnp.int32, (_N, _G), 1)
  onehot = (batch2d[...] == gids).astype(jnp.float32)
  pooled = lax.dot_general(onehot, h2, (((0,), (0,)), ((), ())),
                           preferred_element_type=jnp.float32)
  m = jnp.maximum(jnp.dot(pooled, Wm1[...],
                          preferred_element_type=jnp.float32) + bm1[...], 0.0)
  out_o[...] = jnp.dot(m, Wm2[...],
                       preferred_element_type=jnp.float32) + bm2[...]


_tc1 = pl.pallas_call(
    _tc1_body,
    out_shape=jax.ShapeDtypeStruct((_N, _D), jnp.float32),
)

_tc2 = pl.pallas_call(
    _tc2_body,
    out_shape=(
        jax.ShapeDtypeStruct((_G, 64), jnp.float32),
        jax.ShapeDtypeStruct((_N, 4), jnp.float32),
    ),
)


def kernel(x, W1, b1, g1, be1, W2, b2, W3, b3, g2, be2, W4, b4,
           Wr1, br1, Wr2, br2, Wr3, br3, Wm1, bm1, Wm2, bm2,
           edge_index, batch):
  src = edge_index[0]
  dst = edge_index[1]
  zeros_chunk = jnp.zeros((_RCH, _D), jnp.float32)
  row = lambda v: v.reshape(1, -1)

  p1 = _sc_seg_sum(x, src, dst, zeros_chunk)
  h1 = _tc1(x, p1, W1, row(b1), row(g1), row(be1), W2, row(b2))
  p2 = _sc_seg_sum(h1, src, dst, zeros_chunk)
  out, x_rec = _tc2(h1, p2, W3, row(b3), row(g2), row(be2), W4, row(b4),
                    Wr1, row(br1), Wr2, row(br2), Wr3, row(br3),
                    Wm1, row(bm1), Wm2, row(bm2),
                    batch.reshape(_N, 1))
  return (out, x_rec)


# speedup vs baseline: 8.7172x; 1.8496x over previous
import functools

import jax
import jax.numpy as jnp
from jax import lax
from jax.experimental import pallas as pl
from jax.experimental.pallas import tpu as pltpu
from jax.experimental.pallas import tpu_sc as plsc

_N, _E, _D, _G = 10000, 320000, 128, 64
_HD = _D // 2
_NC, _NS = 2, 16
_EPS = _E // _NS
_ECH = 125
_NCHUNK = _EPS // _ECH
_NP = 10240
_RPS = _NP // _NS
_RCH = 128
_RNCH = _RPS // _RCH


def _sc_body(feat_h, src_h, dst_h, zero_h, out_h,
             src_v, dst_v, rows0, rows1, acc, sem0, sem1):
  c = lax.axis_index("c")
  s = lax.axis_index("s")
  row0 = s * _RPS
  g0 = rows0.at[pl.ds(0, _ECH)]
  g1 = rows1.at[pl.ds(0, _ECH)]

  pltpu.sync_copy(src_h.at[c, s], src_v)
  pltpu.sync_copy(dst_h.at[s], dst_v)

  for k in range(_RNCH):
    rs = row0 + k * _RCH

    @pl.when(rs + _RCH <= _N)
    def _():
      pltpu.sync_copy(feat_h.at[pl.ds(c * _N + rs, _RCH)], rows0)
      pltpu.sync_copy(rows0, acc.at[pl.ds(rs, _RCH)])

  @pl.when(s == _NS - 1)
  def _():
    t16 = rows0.at[pl.ds(0, 16)]
    pltpu.sync_copy(feat_h.at[pl.ds(c * _N + _N - 16, 16)], t16)
    pltpu.sync_copy(t16, acc.at[pl.ds(_N - 16, 16)])
    pltpu.sync_copy(zero_h, rows1)
    pltpu.sync_copy(rows1.at[pl.ds(0, 112)], acc.at[pl.ds(_N, 112)])
    pltpu.sync_copy(rows1, acc.at[pl.ds(_N + 112, _RCH)])

  plsc.subcore_barrier()

  pltpu.async_copy(feat_h.at[src_v.at[0]], g0, sem0)
  pltpu.async_copy(feat_h.at[src_v.at[1]], g1, sem1)

  def step(i, carry):
    j0 = 2 * i
    j1 = j0 + 1
    pltpu.make_async_copy(feat_h.at[src_v.at[j0]], g0, sem0).wait()
    pltpu.sync_copy(g0, acc.at[dst_v.at[j0]], add=True)

    @pl.when(j0 + 2 < _NCHUNK)
    def _():
      pltpu.async_copy(feat_h.at[src_v.at[j0 + 2]], g0, sem0)

    pltpu.make_async_copy(feat_h.at[src_v.at[j1]], g1, sem1).wait()
    pltpu.sync_copy(g1, acc.at[dst_v.at[j1]], add=True)

    @pl.when(j1 + 2 < _NCHUNK)
    def _():
      pltpu.async_copy(feat_h.at[src_v.at[j1 + 2]], g1, sem1)

    return carry

  lax.fori_loop(0, _NCHUNK // 2, step, 0)
  plsc.subcore_barrier()

  for k in range(_RNCH):
    sl = pl.ds(row0 + k * _RCH, _RCH)
    buf = (rows0, rows1)[k % 2]
    pltpu.sync_copy(acc.at[sl], buf)
    pltpu.sync_copy(buf, out_h.at[c, sl])


@functools.cache
def _sc_seg_sum_fn():
  return pl.kernel(
      _sc_body,
      out_type=jax.ShapeDtypeStruct((_NC, _NP, _HD), jnp.float32),
      mesh=plsc.VectorSubcoreMesh(
          core_axis_name="c", subcore_axis_name="s",
          num_cores=_NC, num_subcores=_NS),
      compiler_params=pltpu.CompilerParams(use_tc_tiling_on_sc=False),
      scratch_types=[
          pltpu.VMEM((_NCHUNK, _ECH), jnp.int32),
          pltpu.VMEM((_NCHUNK, _ECH), jnp.int32),
          pltpu.VMEM((_RCH, _HD), jnp.float32),
          pltpu.VMEM((_RCH, _HD), jnp.float32),
          pltpu.VMEM_SHARED((_NP, _HD), jnp.float32),
          pltpu.SemaphoreType.DMA,
          pltpu.SemaphoreType.DMA,
      ],
  )


def _sc_seg_sum(feat_stacked, src2, dst, zeros_chunk):
  return _sc_seg_sum_fn()(feat_stacked, src2, dst, zeros_chunk)


def _gin_mlp(z, W1, b1, g, be, W2, b2):
  h = jnp.dot(z, W1, preferred_element_type=jnp.float32) + b1
  mean = jnp.mean(h, axis=0, keepdims=True)
  var = jnp.mean((h - mean) ** 2, axis=0, keepdims=True)
  h = (h - mean) / jnp.sqrt(var + 1e-5) * g + be
  h = jnp.maximum(h, 0.0)
  h = jnp.dot(h, W2, preferred_element_type=jnp.float32) + b2
  return jnp.maximum(h, 0.0)


def _tc1_body(p, W1, b1, g1, be1, W2, b2, h1_out):
  z = jnp.concatenate([p[0, :_N], p[1, :_N]], axis=1)
  h = _gin_mlp(z, W1[...], b1[...], g1[...], be1[...], W2[...], b2[...])
  h1_out[0:_N] = h[:, :_HD]
  h1_out[_N:2 * _N] = h[:, _HD:]


def _tc2_body(q, W3, b3, g2, be2, W4, b4, Wr1, br1, Wr2, br2, Wr3, br3,
              Wm1, bm1, Wm2, bm2, batch2d, out_o, xrec_o):
  z = jnp.concatenate([q[0, :_N], q[1, :_N]], axis=1)
  h2 = _gin_mlp(z, W3[...], b3[...], g2[...], be2[...], W4[...], b4[...])

  r = jnp.maximum(jnp.dot(h2, Wr1[...],
                          preferred_element_type=jnp.float32) + br1[...], 0.0)
  r = jnp.maximum(jnp.dot(r, Wr2[...],
                          preferred_element_type=jnp.float32) + br2[...], 0.0)
  xrec_o[...] = jnp.maximum(
      jnp.dot(r, Wr3[...], preferred_element_type=jnp.float32) + br3[...], 0.0)

  gids = lax.broadcasted_iota(jnp.int32, (_N, _G), 1)
  onehot = (batch2d[...] == gids).astype(jnp.float32)
  pooled = lax.dot_general(onehot, h2, (((0,), (0,)), ((), ())),
                           preferred_element_type=jnp.float32)
  m = jnp.maximum(jnp.dot(pooled, Wm1[...],
                          preferred_element_type=jnp.float32) + bm1[...], 0.0)
  out_o[...] = jnp.dot(m, Wm2[...],
                       preferred_element_type=jnp.float32) + bm2[...]


_tc1 = pl.pallas_call(
    _tc1_body,
    out_shape=jax.ShapeDtypeStruct((2 * _N, _HD), jnp.float32),
)

_tc2 = pl.pallas_call(
    _tc2_body,
    out_shape=(
        jax.ShapeDtypeStruct((_G, 64), jnp.float32),
        jax.ShapeDtypeStruct((_N, 4), jnp.float32),
    ),
)


def kernel(x, W1, b1, g1, be1, W2, b2, W3, b3, g2, be2, W4, b4,
           Wr1, br1, Wr2, br2, Wr3, br3, Wm1, bm1, Wm2, bm2,
           edge_index, batch):
  src = edge_index[0].reshape(_NS, _NCHUNK, _ECH)
  src2 = jnp.stack([src, src + _N])
  dst = edge_index[1].reshape(_NS, _NCHUNK, _ECH)
  zeros_chunk = jnp.zeros((_RCH, _HD), jnp.float32)
  xs = jnp.concatenate([x[:, :_HD], x[:, _HD:]], axis=0)
  row = lambda v: v.reshape(1, -1)

  p1 = _sc_seg_sum(xs, src2, dst, zeros_chunk)
  h1s = _tc1(p1, W1, row(b1), row(g1), row(be1), W2, row(b2))
  q = _sc_seg_sum(h1s, src2, dst, zeros_chunk)
  out, x_rec = _tc2(q, W3, row(b3), row(g2), row(be2), W4, row(b4),
                    Wr1, row(br1), Wr2, row(br2), Wr3, row(br3),
                    Wm1, row(bm1), Wm2, row(bm2),
                    batch.reshape(_N, 1))
  return (out, x_rec)


# speedup vs baseline: 9.2061x; 1.0561x over previous
import functools

import jax
import jax.numpy as jnp
from jax import lax
from jax.experimental import pallas as pl
from jax.experimental.pallas import tpu as pltpu
from jax.experimental.pallas import tpu_sc as plsc

_N, _E, _D, _G = 10000, 320000, 128, 64
_HD = _D // 2
_NC, _NS = 2, 16
_EPS = _E // _NS
_ECH = 125
_NCHUNK = _EPS // _ECH
_NP = 10240
_RPS = _NP // _NS
_RCH = 128
_RNCH = _RPS // _RCH


def _sc_body(feat_h, src_h, dst_h, zero_h, out_h,
             src_v, dst_v, big, acc,
             gs0, gs1, gs2, gs3, ss0, ss1, ss2, ss3):
  gsem = (gs0, gs1, gs2, gs3)
  ssem = (ss0, ss1, ss2, ss3)
  c = lax.axis_index("c")
  s = lax.axis_index("s")
  row0 = s * _RPS
  gw = tuple(big.at[pl.ds(b * 160, _ECH)] for b in range(4))

  pltpu.sync_copy(src_h.at[c, s], src_v)
  pltpu.sync_copy(dst_h.at[s], dst_v)

  @pl.when(s < _NS - 1)
  def _():
    pltpu.sync_copy(feat_h.at[pl.ds(c * _N + row0, _RPS)], big)

  @pl.when(s == _NS - 1)
  def _():
    pltpu.sync_copy(feat_h.at[pl.ds(c * _N + row0, _N - (_NS - 1) * _RPS)],
                    big.at[pl.ds(0, _N - (_NS - 1) * _RPS)])
    pltpu.sync_copy(zero_h, big.at[pl.ds(_N - (_NS - 1) * _RPS, _NP - _N)])

  pltpu.sync_copy(big, acc.at[pl.ds(row0, _RPS)])
  plsc.subcore_barrier()

  pltpu.async_copy(feat_h.at[src_v.at[0]], gw[0], gsem[0])
  pltpu.async_copy(feat_h.at[src_v.at[1]], gw[1], gsem[1])

  def step(i, carry):
    for b in range(4):
      j = 4 * i + b
      pltpu.make_async_copy(feat_h.at[src_v.at[j]], gw[b], gsem[b]).wait()
      pltpu.async_copy(gw[b], acc.at[dst_v.at[j]], ssem[b], add=True)
      jn = j + 2
      bn = (b + 2) % 4

      @pl.when(jn < _NCHUNK)
      def _(jn=jn, bn=bn):
        @pl.when(jn >= 4)
        def _():
          pltpu.make_async_copy(
              gw[bn], acc.at[dst_v.at[jn - 4]], ssem[bn]).wait()

        pltpu.async_copy(feat_h.at[src_v.at[jn]], gw[bn], gsem[bn])

    return carry

  lax.fori_loop(0, _NCHUNK // 4, step, 0)

  for b in range(4):
    j = _NCHUNK - 4 + b
    pltpu.make_async_copy(gw[b], acc.at[dst_v.at[j]], ssem[b]).wait()
  plsc.subcore_barrier()

  pltpu.sync_copy(acc.at[pl.ds(row0, _RPS)], big)
  pltpu.sync_copy(big, out_h.at[c, pl.ds(row0, _RPS)])


@functools.cache
def _sc_seg_sum_fn():
  return pl.kernel(
      _sc_body,
      out_type=jax.ShapeDtypeStruct((_NC, _NP, _HD), jnp.float32),
      mesh=plsc.VectorSubcoreMesh(
          core_axis_name="c", subcore_axis_name="s",
          num_cores=_NC, num_subcores=_NS),
      compiler_params=pltpu.CompilerParams(use_tc_tiling_on_sc=False),
      scratch_types=[
          pltpu.VMEM((_NCHUNK, _ECH), jnp.int32),
          pltpu.VMEM((_NCHUNK, _ECH), jnp.int32),
          pltpu.VMEM((_RPS, _HD), jnp.float32),
          pltpu.VMEM_SHARED((_NP, _HD), jnp.float32),
          pltpu.SemaphoreType.DMA,
          pltpu.SemaphoreType.DMA,
          pltpu.SemaphoreType.DMA,
          pltpu.SemaphoreType.DMA,
          pltpu.SemaphoreType.DMA,
          pltpu.SemaphoreType.DMA,
          pltpu.SemaphoreType.DMA,
          pltpu.SemaphoreType.DMA,
      ],
  )


def _sc_seg_sum(feat_stacked, src2, dst, zeros_chunk):
  return _sc_seg_sum_fn()(feat_stacked, src2, dst, zeros_chunk)


def _gin_mlp(z, W1, b1, g, be, W2, b2):
  h = jnp.dot(z, W1, preferred_element_type=jnp.float32) + b1
  mean = jnp.mean(h, axis=0, keepdims=True)
  var = jnp.mean((h - mean) ** 2, axis=0, keepdims=True)
  h = (h - mean) / jnp.sqrt(var + 1e-5) * g + be
  h = jnp.maximum(h, 0.0)
  h = jnp.dot(h, W2, preferred_element_type=jnp.float32) + b2
  return jnp.maximum(h, 0.0)


def _tc1_body(p, W1, b1, g1, be1, W2, b2, h1_out):
  z = jnp.concatenate([p[0, :_N], p[1, :_N]], axis=1)
  h = _gin_mlp(z, W1[...], b1[...], g1[...], be1[...], W2[...], b2[...])
  h1_out[0:_N] = h[:, :_HD]
  h1_out[_N:2 * _N] = h[:, _HD:]


def _tc2_body(q, W3, b3, g2, be2, W4, b4, Wr1, br1, Wr2, br2, Wr3, br3,
              Wm1, bm1, Wm2, bm2, batch2d, out_o, xrec_o):
  z = jnp.concatenate([q[0, :_N], q[1, :_N]], axis=1)
  h2 = _gin_mlp(z, W3[...], b3[...], g2[...], be2[...], W4[...], b4[...])

  r = jnp.maximum(jnp.dot(h2, Wr1[...],
                          preferred_element_type=jnp.float32) + br1[...], 0.0)
  r = jnp.maximum(jnp.dot(r, Wr2[...],
                          preferred_element_type=jnp.float32) + br2[...], 0.0)
  xrec_o[...] = jnp.maximum(
      jnp.dot(r, Wr3[...], preferred_element_type=jnp.float32) + br3[...], 0.0)

  gids = lax.broadcasted_iota(jnp.int32, (_N, _G), 1)
  onehot = (batch2d[...] == gids).astype(jnp.float32)
  pooled = lax.dot_general(onehot, h2, (((0,), (0,)), ((), ())),
                           preferred_element_type=jnp.float32)
  m = jnp.maximum(jnp.dot(pooled, Wm1[...],
                          preferred_element_type=jnp.float32) + bm1[...], 0.0)
  out_o[...] = jnp.dot(m, Wm2[...],
                       preferred_element_type=jnp.float32) + bm2[...]


_tc1 = pl.pallas_call(
    _tc1_body,
    out_shape=jax.ShapeDtypeStruct((2 * _N, _HD), jnp.float32),
)

_tc2 = pl.pallas_call(
    _tc2_body,
    out_shape=(
        jax.ShapeDtypeStruct((_G, 64), jnp.float32),
        jax.ShapeDtypeStruct((_N, 4), jnp.float32),
    ),
)


def kernel(x, W1, b1, g1, be1, W2, b2, W3, b3, g2, be2, W4, b4,
           Wr1, br1, Wr2, br2, Wr3, br3, Wm1, bm1, Wm2, bm2,
           edge_index, batch):
  src = edge_index[0].reshape(_NS, _NCHUNK, _ECH)
  src2 = jnp.stack([src, src + _N])
  dst = edge_index[1].reshape(_NS, _NCHUNK, _ECH)
  zeros_chunk = jnp.zeros((_NP - _N, _HD), jnp.float32)
  xs = jnp.concatenate([x[:, :_HD], x[:, _HD:]], axis=0)
  row = lambda v: v.reshape(1, -1)

  p1 = _sc_seg_sum(xs, src2, dst, zeros_chunk)
  h1s = _tc1(p1, W1, row(b1), row(g1), row(be1), W2, row(b2))
  q = _sc_seg_sum(h1s, src2, dst, zeros_chunk)
  out, x_rec = _tc2(q, W3, row(b3), row(g2), row(be2), W4, row(b4),
                    Wr1, row(br1), Wr2, row(br2), Wr3, row(br3),
                    Wm1, row(bm1), Wm2, row(bm2),
                    batch.reshape(_N, 1))
  return (out, x_rec)


# speedup vs baseline: 10.0538x; 1.0921x over previous
import functools

import jax
import jax.numpy as jnp
from jax import lax
from jax.experimental import pallas as pl
from jax.experimental.pallas import tpu as pltpu
from jax.experimental.pallas import tpu_sc as plsc

_N, _E, _D, _G = 10000, 320000, 128, 64
_HD = _D // 2
_NC, _NS = 2, 16
_EPS = _E // _NS
_ECH = 125
_NCHUNK = _EPS // _ECH
_NP = 10240
_RPS = _NP // _NS
_RCH = 128
_RNCH = _RPS // _RCH


_NB = 4
_PF = 3


def _sc_body(feat_h, src_h, dst_h, zero_h, out_h,
             src_v, dst_v, big, acc,
             gs0, gs1, gs2, gs3):
  gsem = (gs0, gs1, gs2, gs3)
  ssem = gsem
  c = lax.axis_index("c")
  s = lax.axis_index("s")
  row0 = s * _RPS
  gw = tuple(big.at[pl.ds(b * _RCH, _ECH)] for b in range(_NB))

  pltpu.sync_copy(src_h.at[c, s], src_v)
  pltpu.sync_copy(dst_h.at[s], dst_v)

  bigr = big.at[pl.ds(0, _RPS)]

  @pl.when(s < _NS - 1)
  def _():
    pltpu.sync_copy(feat_h.at[pl.ds(c * _N + row0, _RPS)], bigr)

  @pl.when(s == _NS - 1)
  def _():
    pltpu.sync_copy(feat_h.at[pl.ds(c * _N + row0, _N - (_NS - 1) * _RPS)],
                    big.at[pl.ds(0, _N - (_NS - 1) * _RPS)])
    pltpu.sync_copy(zero_h, big.at[pl.ds(_N - (_NS - 1) * _RPS, _NP - _N)])

  pltpu.sync_copy(bigr, acc.at[pl.ds(row0, _RPS)])
  plsc.subcore_barrier()

  for b in range(_PF):
    pltpu.async_copy(feat_h.at[src_v.at[b]], gw[b], gsem[b])

  def step(i, carry):
    for b in range(_NB):
      j = _NB * i + b
      pltpu.make_async_copy(feat_h.at[src_v.at[j]], gw[b], gsem[b]).wait()
      pltpu.async_copy(gw[b], acc.at[dst_v.at[j]], ssem[b], add=True)
      jn = j + _PF
      bn = (b + _PF) % _NB

      @pl.when(jn < _NCHUNK)
      def _(jn=jn, bn=bn):
        @pl.when(jn >= _NB)
        def _():
          pltpu.make_async_copy(
              gw[bn], acc.at[dst_v.at[jn - _NB]], ssem[bn]).wait()

        pltpu.async_copy(feat_h.at[src_v.at[jn]], gw[bn], gsem[bn])

    return carry

  lax.fori_loop(0, _NCHUNK // _NB, step, 0)

  for b in range(_NB):
    j = _NCHUNK - _NB + b
    pltpu.make_async_copy(gw[b], acc.at[dst_v.at[j]], ssem[b]).wait()
  plsc.subcore_barrier()

  pltpu.sync_copy(acc.at[pl.ds(row0, _RPS)], bigr)
  pltpu.sync_copy(bigr, out_h.at[c, pl.ds(row0, _RPS)])


@functools.cache
def _sc_seg_sum_fn():
  return pl.kernel(
      _sc_body,
      out_type=jax.ShapeDtypeStruct((_NC, _NP, _HD), jnp.float32),
      mesh=plsc.VectorSubcoreMesh(
          core_axis_name="c", subcore_axis_name="s",
          num_cores=_NC, num_subcores=_NS),
      compiler_params=pltpu.CompilerParams(use_tc_tiling_on_sc=False),
      scratch_types=[
          pltpu.VMEM((_NCHUNK, _ECH), jnp.int32),
          pltpu.VMEM((_NCHUNK, _ECH), jnp.int32),
          pltpu.VMEM((max(_NB * _RCH, _RPS), _HD), jnp.float32),
          pltpu.VMEM_SHARED((_NP, _HD), jnp.float32),
      ] + [pltpu.SemaphoreType.DMA] * _NB,
  )


def _sc_seg_sum(feat_stacked, src2, dst, zeros_chunk):
  return _sc_seg_sum_fn()(feat_stacked, src2, dst, zeros_chunk)


def _gin_mlp(z, W1, b1, g, be, W2, b2):
  h = jnp.dot(z, W1, preferred_element_type=jnp.float32) + b1
  mean = jnp.mean(h, axis=0, keepdims=True)
  var = jnp.mean((h - mean) ** 2, axis=0, keepdims=True)
  h = (h - mean) / jnp.sqrt(var + 1e-5) * g + be
  h = jnp.maximum(h, 0.0)
  h = jnp.dot(h, W2, preferred_element_type=jnp.float32) + b2
  return jnp.maximum(h, 0.0)


def _tc1_body(p, W1, b1, g1, be1, W2, b2, h1_out):
  z = jnp.concatenate([p[0, :_N], p[1, :_N]], axis=1)
  h = _gin_mlp(z, W1[...], b1[...], g1[...], be1[...], W2[...], b2[...])
  h1_out[0:_N] = h[:, :_HD]
  h1_out[_N:2 * _N] = h[:, _HD:]


def _tc2_body(q, W3, b3, g2, be2, W4, b4, Wr1, br1, Wr2, br2, Wr3, br3,
              Wm1, bm1, Wm2, bm2, batch2d, out_o, xrec_o):
  z = jnp.concatenate([q[0, :_N], q[1, :_N]], axis=1)
  h2 = _gin_mlp(z, W3[...], b3[...], g2[...], be2[...], W4[...], b4[...])

  r = jnp.maximum(jnp.dot(h2, Wr1[...],
                          preferred_element_type=jnp.float32) + br1[...], 0.0)
  r = jnp.maximum(jnp.dot(r, Wr2[...],
                          preferred_element_type=jnp.float32) + br2[...], 0.0)
  xrec_o[...] = jnp.maximum(
      jnp.dot(r, Wr3[...], preferred_element_type=jnp.float32) + br3[...], 0.0)

  gids = lax.broadcasted_iota(jnp.int32, (_N, _G), 1)
  onehot = (batch2d[...] == gids).astype(jnp.float32)
  pooled = lax.dot_general(onehot, h2, (((0,), (0,)), ((), ())),
                           preferred_element_type=jnp.float32)
  m = jnp.maximum(jnp.dot(pooled, Wm1[...],
                          preferred_element_type=jnp.float32) + bm1[...], 0.0)
  out_o[...] = jnp.dot(m, Wm2[...],
                       preferred_element_type=jnp.float32) + bm2[...]


_tc1 = pl.pallas_call(
    _tc1_body,
    out_shape=jax.ShapeDtypeStruct((2 * _N, _HD), jnp.float32),
)

_tc2 = pl.pallas_call(
    _tc2_body,
    out_shape=(
        jax.ShapeDtypeStruct((_G, 64), jnp.float32),
        jax.ShapeDtypeStruct((_N, 4), jnp.float32),
    ),
)


def kernel(x, W1, b1, g1, be1, W2, b2, W3, b3, g2, be2, W4, b4,
           Wr1, br1, Wr2, br2, Wr3, br3, Wm1, bm1, Wm2, bm2,
           edge_index, batch):
  src = edge_index[0].reshape(_NS, _NCHUNK, _ECH)
  src2 = jnp.stack([src, src + _N])
  dst = edge_index[1].reshape(_NS, _NCHUNK, _ECH)
  zeros_chunk = jnp.zeros((_NP - _N, _HD), jnp.float32)
  xs = jnp.concatenate([x[:, :_HD], x[:, _HD:]], axis=0)
  row = lambda v: v.reshape(1, -1)

  p1 = _sc_seg_sum(xs, src2, dst, zeros_chunk)
  h1s = _tc1(p1, W1, row(b1), row(g1), row(be1), W2, row(b2))
  q = _sc_seg_sum(h1s, src2, dst, zeros_chunk)
  out, x_rec = _tc2(q, W3, row(b3), row(g2), row(be2), W4, row(b4),
                    Wr1, row(br1), Wr2, row(br2), Wr3, row(br3),
                    Wm1, row(bm1), Wm2, row(bm2),
                    batch.reshape(_N, 1))
  return (out, x_rec)


# speedup vs baseline: 10.8865x; 1.0828x over previous
import functools

import jax
import jax.numpy as jnp
from jax import lax
from jax.experimental import pallas as pl
from jax.experimental.pallas import tpu as pltpu
from jax.experimental.pallas import tpu_sc as plsc

_N, _E, _D, _G = 10000, 320000, 128, 64
_HD = _D // 2
_NC, _NS = 2, 16
_EPS = _E // _NS
_ECH = 125
_NCHUNK = _EPS // _ECH
_NP = 10240
_RPS = _NP // _NS
_RCH = 128
_RNCH = _RPS // _RCH


_NB = 4
_PF = 3


def _sc_body(feat_h, src_h, dst_h, zero_h, out_h,
             src_v, dst_v, big, acc,
             gs0, gs1, gs2, gs3):
  gsem = (gs0, gs1, gs2, gs3)
  ssem = gsem
  c = lax.axis_index("c")
  s = lax.axis_index("s")
  row0 = s * _RPS
  gw = tuple(big.at[pl.ds(b * _RCH, _ECH)] for b in range(_NB))

  pltpu.sync_copy(src_h.at[c, s], src_v)
  pltpu.sync_copy(dst_h.at[s], dst_v)

  bigr = big.at[pl.ds(0, _RPS)]
  pltpu.sync_copy(zero_h, bigr)
  pltpu.sync_copy(bigr, acc.at[pl.ds(row0, _RPS)])
  plsc.subcore_barrier()

  for b in range(_PF):
    pltpu.async_copy(feat_h.at[src_v.at[b]], gw[b], gsem[b])

  def step(i, carry):
    for b in range(_NB):
      j = _NB * i + b
      pltpu.make_async_copy(feat_h.at[src_v.at[j]], gw[b], gsem[b]).wait()
      pltpu.async_copy(gw[b], acc.at[dst_v.at[j]], ssem[b], add=True)
      jn = j + _PF
      bn = (b + _PF) % _NB

      @pl.when(jn < _NCHUNK)
      def _(jn=jn, bn=bn):
        @pl.when(jn >= _NB)
        def _():
          pltpu.make_async_copy(
              gw[bn], acc.at[dst_v.at[jn - _NB]], ssem[bn]).wait()

        pltpu.async_copy(feat_h.at[src_v.at[jn]], gw[bn], gsem[bn])

    return carry

  lax.fori_loop(0, _NCHUNK // _NB, step, 0)

  for b in range(_NB):
    j = _NCHUNK - _NB + b
    pltpu.make_async_copy(gw[b], acc.at[dst_v.at[j]], ssem[b]).wait()
  plsc.subcore_barrier()

  pltpu.sync_copy(acc.at[pl.ds(row0, _RPS)], bigr)
  pltpu.sync_copy(bigr, out_h.at[c, pl.ds(row0, _RPS)])


@functools.cache
def _sc_seg_sum_fn():
  return pl.kernel(
      _sc_body,
      out_type=jax.ShapeDtypeStruct((_NC, _NP, _HD), jnp.float32),
      mesh=plsc.VectorSubcoreMesh(
          core_axis_name="c", subcore_axis_name="s",
          num_cores=_NC, num_subcores=_NS),
      compiler_params=pltpu.CompilerParams(use_tc_tiling_on_sc=False),
      scratch_types=[
          pltpu.VMEM((_NCHUNK, _ECH), jnp.int32),
          pltpu.VMEM((_NCHUNK, _ECH), jnp.int32),
          pltpu.VMEM((max(_NB * _RCH, _RPS), _HD), jnp.float32),
          pltpu.VMEM_SHARED((_NP, _HD), jnp.float32),
      ] + [pltpu.SemaphoreType.DMA] * _NB,
  )


def _sc_seg_sum(feat_stacked, src2, dst, zeros_chunk):
  return _sc_seg_sum_fn()(feat_stacked, src2, dst, zeros_chunk)


def _gin_mlp(z, W1, b1, g, be, W2, b2):
  h = jnp.dot(z, W1, preferred_element_type=jnp.float32) + b1
  mean = jnp.mean(h, axis=0, keepdims=True)
  var = jnp.mean((h - mean) ** 2, axis=0, keepdims=True)
  h = (h - mean) / jnp.sqrt(var + 1e-5) * g + be
  h = jnp.maximum(h, 0.0)
  h = jnp.dot(h, W2, preferred_element_type=jnp.float32) + b2
  return jnp.maximum(h, 0.0)


def _tc1_body(x, p, W1, b1, g1, be1, W2, b2, h1_out):
  z = x[...] + jnp.concatenate([p[0, :_N], p[1, :_N]], axis=1)
  h1_out[...] = _gin_mlp(z, W1[...], b1[...], g1[...], be1[...],
                         W2[...], b2[...])


def _tc2_body(h1, q, W3, b3, g2, be2, W4, b4, Wr1, br1, Wr2, br2, Wr3, br3,
              Wm1, bm1, Wm2, bm2, batch2d, out_o, xrec_o):
  z = h1[...] + jnp.concatenate([q[0, :_N], q[1, :_N]], axis=1)
  h2 = _gin_mlp(z, W3[...], b3[...], g2[...], be2[...], W4[...], b4[...])

  r = jnp.maximum(jnp.dot(h2, Wr1[...],
                          preferred_element_type=jnp.float32) + br1[...], 0.0)
  r = jnp.maximum(jnp.dot(r, Wr2[...],
                          preferred_element_type=jnp.float32) + br2[...], 0.0)
  xrec_o[...] = jnp.maximum(
      jnp.dot(r, Wr3[...], preferred_element_type=jnp.float32) + br3[...], 0.0)

  gids = lax.broadcasted_iota(jnp.int32, (_N, _G), 1)
  onehot = (batch2d[...] == gids).astype(jnp.float32)
  pooled = lax.dot_general(onehot, h2, (((0,), (0,)), ((), ())),
                           preferred_element_type=jnp.float32)
  m = jnp.maximum(jnp.dot(pooled, Wm1[...],
                          preferred_element_type=jnp.float32) + bm1[...], 0.0)
  out_o[...] = jnp.dot(m, Wm2[...],
                       preferred_element_type=jnp.float32) + bm2[...]


_tc1 = pl.pallas_call(
    _tc1_body,
    out_shape=jax.ShapeDtypeStruct((_N, _D), jnp.float32),
)

_tc2 = pl.pallas_call(
    _tc2_body,
    out_shape=(
        jax.ShapeDtypeStruct((_G, 64), jnp.float32),
        jax.ShapeDtypeStruct((_N, 4), jnp.float32),
    ),
)


def kernel(x, W1, b1, g1, be1, W2, b2, W3, b3, g2, be2, W4, b4,
           Wr1, br1, Wr2, br2, Wr3, br3, Wm1, bm1, Wm2, bm2,
           edge_index, batch):
  src = edge_index[0].reshape(_NS, _NCHUNK, _ECH)
  src2 = jnp.stack([2 * src, 2 * src + 1])
  dst = edge_index[1].reshape(_NS, _NCHUNK, _ECH)
  zeros_chunk = jnp.zeros((_RPS, _HD), jnp.float32)
  row = lambda v: v.reshape(1, -1)

  p1 = _sc_seg_sum(x.reshape(2 * _N, _HD), src2, dst, zeros_chunk)
  h1 = _tc1(x, p1, W1, row(b1), row(g1), row(be1), W2, row(b2))
  q = _sc_seg_sum(h1.reshape(2 * _N, _HD), src2, dst, zeros_chunk)
  out, x_rec = _tc2(h1, q, W3, row(b3), row(g2), row(be2), W4, row(b4),
                    Wr1, row(br1), Wr2, row(br2), Wr3, row(br3),
                    Wm1, row(bm1), Wm2, row(bm2),
                    batch.reshape(_N, 1))
  return (out, x_rec)
